# Initial kernel scaffold; baseline (speedup 1.0000x reference)
import jax
import jax.numpy as jnp
from jax.experimental import pallas as pl


def kernel(x, d, weight1, bias1, weight2, bias2, weight3, bias3, weight4, bias4, weight5, bias5):
    raise NotImplementedError("write your pallas kernel here")


# speedup vs baseline: 1.1933x; 1.1933x over previous
import functools

import jax
import jax.numpy as jnp
from jax.experimental import pallas as pl
from jax.experimental.pallas import tpu as pltpu

_N = 16
_L_LOC = 10
_L_DIR = 4
_SCALE = 3.0
_K = 256
_NCELLS = _N ** 3


def _encode(v, L):
    parts = [v]
    for j in range(L):
        s = (2.0 ** j) * v
        parts.append(jnp.sin(s))
        parts.append(jnp.cos(s))
    return jnp.concatenate(parts, axis=1)


def _mlp_kernel(blk_ref, cell_ref, cellw_ref, first_ref, nit_ref,
                ex_ref, ed_ref, scid_ref,
                w1_ref, b1_ref, w2a_ref, b2a_ref, w2b_ref, b2b_ref,
                w3_ref, b3_ref, w4a_ref, w4b_ref, b4_ref,
                w5_ref, b5_ref,
                color_ref, dens_ref):
    g = pl.program_id(0)

    @pl.when(g < nit_ref[0])
    def _run():
        cell = cell_ref[g]
        dot = functools.partial(jnp.dot, preferred_element_type=jnp.float32)
        ex = ex_ref[0]
        ed = ed_ref[0]
        h1 = jnp.maximum(dot(ex, w1_ref[0]) + b1_ref[0], 0.0)
        sigma = jnp.maximum(dot(h1, w2a_ref[0]) + b2a_ref[0], 0.0)
        h2 = jnp.maximum(dot(h1, w2b_ref[0]) + b2b_ref[0], 0.0)
        h3 = dot(h2, w3_ref[0]) + b3_ref[0]
        h4 = jnp.maximum(dot(h3, w4a_ref[0]) + dot(ed, w4b_ref[0]) + b4_ref[0], 0.0)
        c = jax.nn.sigmoid(dot(h4, w5_ref[0]) + b5_ref[0])
        m2 = (scid_ref[0] == cell) & (cell < _NCELLS)
        first = first_ref[g] == 1

        @pl.when(first)
        def _init():
            color_ref[0] = jnp.where(m2, c, 0.0)
            dens_ref[0] = jnp.where(m2, sigma, 0.0)

        @pl.when(jnp.logical_not(first))
        def _acc():
            color_ref[0] = jnp.where(m2, c, color_ref[0])
            dens_ref[0] = jnp.where(m2, sigma, dens_ref[0])


def kernel(x, d, weight1, bias1, weight2, bias2, weight3, bias3, weight4,
           bias4, weight5, bias5):
    B = x.shape[0]
    nblk = B // _K
    G = nblk + _NCELLS

    mask = ((jnp.abs(x[:, 0]) < _SCALE / 2)
            & (jnp.abs(x[:, 1]) < _SCALE / 2)
            & (jnp.abs(x[:, 2]) < _SCALE / 2))
    i = jnp.clip((x / (_SCALE / _N) + _N / 2).astype(jnp.int32), 0, _N - 1)
    cid = (i[:, 0] * _N + i[:, 1]) * _N + i[:, 2]
    cid = jnp.where(mask, cid, _NCELLS)

    order = jnp.argsort(cid)
    scid = cid[order]
    xs = x[order]
    ds = d[order]
    ex = _encode(xs, _L_LOC).reshape(nblk, _K, 6 * _L_LOC + 3)
    ed = _encode(ds, _L_DIR).reshape(nblk, _K, 6 * _L_DIR + 3)
    scid3 = scid.reshape(nblk, _K, 1)

    p = jnp.arange(B, dtype=jnp.int32)
    changed = jnp.concatenate(
        [jnp.ones((1,), jnp.bool_), scid[1:] != scid[:-1]])
    flags = ((p % _K) == 0) | changed
    nitems = jnp.sum(flags).astype(jnp.int32).reshape(1)
    item_pos = jnp.nonzero(flags, size=G, fill_value=B - 1)[0].astype(jnp.int32)
    item_blk = item_pos // _K
    item_cell = scid[item_pos]
    item_cellw = jnp.minimum(item_cell, _NCELLS - 1)
    item_first = (item_pos % _K == 0).astype(jnp.int32)

    w1r = weight1.reshape(_NCELLS, 63, 32)
    b1r = bias1.reshape(_NCELLS, 1, 32)
    w2 = weight2.reshape(_NCELLS, 32, 33)
    w2a = w2[:, :, 0:1]
    w2b = w2[:, :, 1:33]
    b2 = bias2.reshape(_NCELLS, 1, 33)
    b2a = b2[:, :, 0:1]
    b2b = b2[:, :, 1:33]
    w3r = weight3.reshape(_NCELLS, 32, 32)
    b3r = bias3.reshape(_NCELLS, 1, 32)
    w4 = weight4.reshape(_NCELLS, 59, 32)
    w4a = w4[:, 0:32, :]
    w4b = w4[:, 32:59, :]
    b4r = bias4.reshape(_NCELLS, 1, 32)
    w5r = weight5.reshape(_NCELLS, 32, 3)
    b5r = bias5.reshape(_NCELLS, 1, 3)

    def im_blk(g, blk, cell, cellw, first, nit):
        return (blk[g], 0, 0)

    def im_cell(g, blk, cell, cellw, first, nit):
        return (cellw[g], 0, 0)

    grid_spec = pltpu.PrefetchScalarGridSpec(
        num_scalar_prefetch=5,
        grid=(G,),
        in_specs=[
            pl.BlockSpec((1, _K, 63), im_blk),
            pl.BlockSpec((1, _K, 27), im_blk),
            pl.BlockSpec((1, _K, 1), im_blk),
            pl.BlockSpec((1, 63, 32), im_cell),
            pl.BlockSpec((1, 1, 32), im_cell),
            pl.BlockSpec((1, 32, 1), im_cell),
            pl.BlockSpec((1, 1, 1), im_cell),
            pl.BlockSpec((1, 32, 32), im_cell),
            pl.BlockSpec((1, 1, 32), im_cell),
            pl.BlockSpec((1, 32, 32), im_cell),
            pl.BlockSpec((1, 1, 32), im_cell),
            pl.BlockSpec((1, 32, 32), im_cell),
            pl.BlockSpec((1, 27, 32), im_cell),
            pl.BlockSpec((1, 1, 32), im_cell),
            pl.BlockSpec((1, 32, 3), im_cell),
            pl.BlockSpec((1, 1, 3), im_cell),
        ],
        out_specs=[
            pl.BlockSpec((1, _K, 3), im_blk),
            pl.BlockSpec((1, _K, 1), im_blk),
        ],
    )
    color_s, dens_s = pl.pallas_call(
        _mlp_kernel,
        grid_spec=grid_spec,
        out_shape=[
            jax.ShapeDtypeStruct((nblk, _K, 3), jnp.float32),
            jax.ShapeDtypeStruct((nblk, _K, 1), jnp.float32),
        ],
    )(item_blk, item_cell, item_cellw, item_first, nitems,
      ex, ed, scid3,
      w1r, b1r, w2a, b2a, w2b, b2b, w3r, b3r, w4a, w4b, b4r, w5r, b5r)

    color = jnp.zeros((B, 3), jnp.float32).at[order].set(color_s.reshape(B, 3))
    density = jnp.zeros((B, 1), jnp.float32).at[order].set(dens_s.reshape(B, 1))
    return (color, density)
